# baseline (speedup 1.0000x reference)
import functools

import jax
import jax.numpy as jnp
from jax import lax
from jax.experimental import pallas as pl
from jax.experimental.pallas import tpu as pltpu
from jax.experimental.pallas import tpu_sc as plsc


def _make_emb_kernel(B, D, NW, n_ch, CH, NB):
    b_per_w = B // NW
    n_t = n_ch // NB
    mesh = plsc.VectorSubcoreMesh(core_axis_name="c", subcore_axis_name="s")

    scratch = [pltpu.VMEM((n_ch, CH), jnp.int32)]
    scratch += [pltpu.VMEM((CH, D), jnp.float32) for _ in range(NB)]
    scratch += [pltpu.SemaphoreType.DMA for _ in range(2 * NB)]

    @functools.partial(
        pl.kernel,
        mesh=mesh,
        out_type=jax.ShapeDtypeStruct((B, D), jnp.float32),
        scratch_types=scratch,
    )
    def emb(table_hbm, idx_hbm, out_hbm, idx_v, *rest):
        bufs = rest[:NB]
        gsem = rest[NB:2 * NB]
        osem = rest[2 * NB:]
        wid = lax.axis_index("s") * 2 + lax.axis_index("c")
        base = wid * b_per_w
        pltpu.sync_copy(idx_hbm.at[wid], idx_v)

        def gather_start(j, b):
            pltpu.async_copy(table_hbm.at[idx_v.at[j]], bufs[b], gsem[b])

        def gather_wait(b):
            pltpu.make_async_copy(
                table_hbm.at[idx_v.at[0]], bufs[b], gsem[b]).wait()

        def write_start(j, b):
            pltpu.async_copy(
                bufs[b], out_hbm.at[pl.ds(base + j * CH, CH)], osem[b])

        def write_wait(b):
            pltpu.make_async_copy(
                bufs[b], out_hbm.at[pl.ds(base, CH)], osem[b]).wait()

        for b in range(NB):
            gather_start(b, b)

        def step(j, b):
            gather_wait(b)
            write_start(j, b)
            bp = (b - 1) % NB
            write_wait(bp)
            gather_start(j - 1 + NB, bp)

        gather_wait(0)
        write_start(0, 0)
        for b in range(1, NB):
            step(b, b)

        def body(t, carry):
            for b in range(NB):
                step(t * NB + b, b)
            return carry

        lax.fori_loop(1, n_t - 1, body, 0)

        j0 = (n_t - 1) * NB
        step(j0, 0)
        for b in range(1, NB):
            gather_wait(b)
            write_start(j0 + b, b)
        for b in range(NB):
            write_wait(b)

    return emb


def kernel(x, table):
    B0, B1 = x.shape
    B = B0 * B1
    D = table.shape[1]
    info = plsc.get_sparse_core_info()
    NW = info.num_cores * info.num_subcores
    CH = 128
    NB = 2
    b_per_w = B // NW
    n_ch = b_per_w // CH
    idx = x.T.reshape(NW, n_ch, CH).astype(jnp.int32)
    out = _make_emb_kernel(B, D, NW, n_ch, CH, NB)(table, idx)
    return out.reshape(B1, B0, D).transpose(1, 0, 2)


# speedup vs baseline: 1.2068x; 1.2068x over previous
import functools

import jax
import jax.numpy as jnp
from jax import lax
from jax.experimental import pallas as pl
from jax.experimental.pallas import tpu as pltpu
from jax.experimental.pallas import tpu_sc as plsc


def _make_emb_kernel(B, D, NW, n_ch, CH, NB):
    b_per_w = B // NW
    n_t = n_ch // NB
    mesh = plsc.VectorSubcoreMesh(core_axis_name="c", subcore_axis_name="s")

    scratch = [pltpu.VMEM((n_ch, CH), jnp.int32)]
    scratch += [pltpu.VMEM((CH, D), jnp.float32) for _ in range(NB)]
    scratch += [pltpu.SemaphoreType.DMA for _ in range(2 * NB)]

    @functools.partial(
        pl.kernel,
        mesh=mesh,
        out_type=jax.ShapeDtypeStruct((B, D), jnp.float32),
        scratch_types=scratch,
    )
    def emb(table_hbm, idx_hbm, out_hbm, idx_v, *rest):
        bufs = rest[:NB]
        gsem = rest[NB:2 * NB]
        osem = rest[2 * NB:]
        wid = lax.axis_index("s") * 2 + lax.axis_index("c")
        base = wid * b_per_w
        pltpu.sync_copy(idx_hbm.at[wid], idx_v)

        def gather_start(j, b):
            pltpu.async_copy(table_hbm.at[idx_v.at[j]], bufs[b], gsem[b])

        def gather_wait(b):
            pltpu.make_async_copy(
                table_hbm.at[idx_v.at[0]], bufs[b], gsem[b]).wait()

        def write_start(j, b):
            pltpu.async_copy(
                bufs[b], out_hbm.at[pl.ds(base + j * CH, CH)], osem[b])

        def write_wait(b):
            pltpu.make_async_copy(
                bufs[b], out_hbm.at[pl.ds(base, CH)], osem[b]).wait()

        for b in range(NB):
            gather_start(b, b)

        def step(j, b):
            gather_wait(b)
            write_start(j, b)
            bp = (b - 1) % NB
            write_wait(bp)
            gather_start(j - 1 + NB, bp)

        gather_wait(0)
        write_start(0, 0)
        for b in range(1, NB):
            step(b, b)

        def body(t, carry):
            for b in range(NB):
                step(t * NB + b, b)
            return carry

        lax.fori_loop(1, n_t - 1, body, 0)

        j0 = (n_t - 1) * NB
        step(j0, 0)
        for b in range(1, NB):
            gather_wait(b)
            write_start(j0 + b, b)
        for b in range(NB):
            write_wait(b)

    return emb


def kernel(x, table):
    B0, B1 = x.shape
    B = B0 * B1
    D = table.shape[1]
    info = plsc.get_sparse_core_info()
    NW = info.num_cores * info.num_subcores
    CH = 128
    NB = 5
    b_per_w = B // NW
    n_ch = b_per_w // CH
    idx = x.T.reshape(NW, n_ch, CH).astype(jnp.int32)
    out = _make_emb_kernel(B, D, NW, n_ch, CH, NB)(table, idx)
    return out.reshape(B1, B0, D).transpose(1, 0, 2)


# speedup vs baseline: 1.2146x; 1.0065x over previous
import functools

import jax
import jax.numpy as jnp
from jax import lax
from jax.experimental import pallas as pl
from jax.experimental.pallas import tpu as pltpu
from jax.experimental.pallas import tpu_sc as plsc


def _make_emb_kernel(B, D, NW, n_ch, CH, NB):
    b_per_w = B // NW
    n_t = n_ch // NB
    mesh = plsc.VectorSubcoreMesh(core_axis_name="c", subcore_axis_name="s")

    scratch = [pltpu.VMEM((n_ch, CH), jnp.int32)]
    scratch += [pltpu.VMEM((CH, D), jnp.float32) for _ in range(NB)]
    scratch += [pltpu.SemaphoreType.DMA for _ in range(2 * NB)]

    @functools.partial(
        pl.kernel,
        mesh=mesh,
        out_type=jax.ShapeDtypeStruct((B, D), jnp.float32),
        scratch_types=scratch,
    )
    def emb(table_hbm, idx_hbm, out_hbm, idx_v, *rest):
        bufs = rest[:NB]
        gsem = rest[NB:2 * NB]
        osem = rest[2 * NB:]
        wid = lax.axis_index("s") * 2 + lax.axis_index("c")
        base = wid * b_per_w
        pltpu.sync_copy(idx_hbm.at[wid], idx_v)

        def gather_start(j, b):
            pltpu.async_copy(table_hbm.at[idx_v.at[j]], bufs[b], gsem[b])

        def gather_wait(b):
            pltpu.make_async_copy(
                table_hbm.at[idx_v.at[0]], bufs[b], gsem[b]).wait()

        def write_start(j, b):
            pltpu.async_copy(
                bufs[b], out_hbm.at[pl.ds(base + j * CH, CH)], osem[b])

        def write_wait(b):
            pltpu.make_async_copy(
                bufs[b], out_hbm.at[pl.ds(base, CH)], osem[b]).wait()

        for b in range(NB):
            gather_start(b, b)

        def step(j, b):
            gather_wait(b)
            write_start(j, b)
            bp = (b - 1) % NB
            write_wait(bp)
            gather_start(j - 1 + NB, bp)

        gather_wait(0)
        write_start(0, 0)
        for b in range(1, NB):
            step(b, b)

        def body(t, carry):
            for b in range(NB):
                step(t * NB + b, b)
            return carry

        lax.fori_loop(1, n_t - 1, body, 0)

        j0 = (n_t - 1) * NB
        step(j0, 0)
        for b in range(1, NB):
            gather_wait(b)
            write_start(j0 + b, b)
        for b in range(NB):
            write_wait(b)

    return emb


def kernel(x, table):
    B0, B1 = x.shape
    B = B0 * B1
    D = table.shape[1]
    info = plsc.get_sparse_core_info()
    NW = info.num_cores * info.num_subcores
    CH = 64
    NB = 10
    b_per_w = B // NW
    n_ch = b_per_w // CH
    idx = x.T.reshape(NW, n_ch, CH).astype(jnp.int32)
    out = _make_emb_kernel(B, D, NW, n_ch, CH, NB)(table, idx)
    return out.reshape(B1, B0, D).transpose(1, 0, 2)
